# baseline (speedup 1.0000x reference)
import functools

import jax
import jax.numpy as jnp
from jax import lax
from jax.experimental import pallas as pl
from jax.experimental.pallas import tpu as pltpu
from jax.experimental.pallas import tpu_sc as plsc

_K = 197
_KP = 256
_DISC_COL = _KP + _K
_D = 2 * _KP
_NW = 32
_CH = 128


def _sc_gather(tbl, idx):
    B = idx.shape[0]
    D = tbl.shape[1]
    bpw = B // _NW
    mesh = plsc.VectorSubcoreMesh(core_axis_name="c", subcore_axis_name="s")

    @functools.partial(
        pl.kernel,
        mesh=mesh,
        out_type=jax.ShapeDtypeStruct((B, D), jnp.float32),
        scratch_types=[
            pltpu.VMEM((_CH,), jnp.int32),
            pltpu.VMEM((_CH, D), jnp.float32),
            pltpu.SemaphoreType.DMA,
        ],
    )
    def k(tbl_hbm, idx_hbm, out_hbm, idx_v, rows_v, sem):
        wid = lax.axis_index("s") * 2 + lax.axis_index("c")
        base = wid * bpw
        for ci in range(bpw // _CH):
            off = base + ci * _CH
            pltpu.sync_copy(idx_hbm.at[pl.ds(off, _CH)], idx_v)
            pltpu.async_copy(tbl_hbm.at[idx_v], rows_v, sem).wait()
            pltpu.sync_copy(rows_v, out_hbm.at[pl.ds(off, _CH)])

    return k(tbl, idx)


def _mlp_body(g_ref, m_ref, sid_ref, semb_ref,
              w1_ref, b1_ref, w2_ref, b2_ref, w3_ref, b3_ref,
              out_ref, ekp_ref):
    bb = m_ref.shape[0]
    stu_n = semb_ref.shape[0]
    ids = sid_ref[...]
    row = lax.broadcasted_iota(jnp.int32, (bb, stu_n), 1)
    oh = (ids == row).astype(jnp.float32)
    stu = jnp.dot(oh, semb_ref[...], preferred_element_type=jnp.float32)
    stat = jax.nn.sigmoid(stu)

    kd = jax.nn.sigmoid(g_ref[:, :_K])
    ekp = g_ref[:, _KP:_KP + _K]
    ekp_ref[...] = ekp
    disc = jax.nn.sigmoid(g_ref[:, _DISC_COL:_DISC_COL + 1]) * 10.0

    x = disc * (stat - kd) * (m_ref[...] * jax.nn.sigmoid(ekp))
    h1 = jax.nn.sigmoid(
        jnp.dot(x.astype(jnp.bfloat16), w1_ref[...].astype(jnp.bfloat16),
                preferred_element_type=jnp.float32)
        + b1_ref[...])
    h2 = jax.nn.sigmoid(
        jnp.dot(h1.astype(jnp.bfloat16), w2_ref[...].astype(jnp.bfloat16),
                preferred_element_type=jnp.float32)
        + b2_ref[...])
    p = jax.nn.sigmoid(
        jnp.dot(h2, w3_ref[...], preferred_element_type=jnp.float32)
        + b3_ref[...])
    out_ref[:, 0:1] = 1.0 - p
    out_ref[:, 1:2] = p


def _tc_mlp(gathered, masks, sid2, student_emb, w1t, b1r, w2t, b2r, w3t, b3r):
    B = masks.shape[0]
    BB = 1024
    grid = (B // BB,)
    stu_n, k = student_emb.shape
    l1 = w1t.shape[1]
    l2 = w2t.shape[1]
    full = lambda shp: pl.BlockSpec(shp, lambda i: (0, 0))
    return pl.pallas_call(
        _mlp_body,
        grid=grid,
        in_specs=[
            pl.BlockSpec((BB, _D), lambda i: (i, 0)),
            pl.BlockSpec((BB, k), lambda i: (i, 0)),
            pl.BlockSpec((BB, 1), lambda i: (i, 0)),
            full((stu_n, k)),
            full((k, l1)), full((1, l1)),
            full((l1, l2)), full((1, l2)),
            full((l2, 1)), full((1, 1)),
        ],
        out_specs=[
            pl.BlockSpec((BB, 2), lambda i: (i, 0)),
            pl.BlockSpec((BB, k), lambda i: (i, 0)),
        ],
        out_shape=[
            jax.ShapeDtypeStruct((B, 2), jnp.float32),
            jax.ShapeDtypeStruct((B, k), jnp.float32),
        ],
    )(gathered, masks, sid2, student_emb,
      w1t, b1r, w2t, b2r, w3t, b3r)


def kernel(stu_id, input_exercise, knowledge_masks, student_emb, k_difficulty,
           e_discrimination, e_k_prob, W1, b1, W2, b2, W3, b3):
    exer_n, k = k_difficulty.shape
    z = jnp.zeros((exer_n, _KP - k), jnp.float32)
    tbl = jnp.concatenate(
        [k_difficulty, z, e_k_prob, e_discrimination,
         jnp.zeros((exer_n, _D - _DISC_COL - 1), jnp.float32)],
        axis=1)

    gathered = _sc_gather(tbl, input_exercise.astype(jnp.int32))

    out, ekp = _tc_mlp(
        gathered, knowledge_masks, stu_id.astype(jnp.int32).reshape(-1, 1),
        student_emb, W1.T, b1.reshape(1, -1), W2.T, b2.reshape(1, -1),
        W3.T, b3.reshape(1, -1))
    return (out, ekp)


# speedup vs baseline: 1.0696x; 1.0696x over previous
import functools

import jax
import jax.numpy as jnp
from jax import lax
from jax.experimental import pallas as pl
from jax.experimental.pallas import tpu as pltpu
from jax.experimental.pallas import tpu_sc as plsc

_K = 197
_EKP_COL = _K
_DISC_COL = 2 * _K
_D = 512
_NW = 32
_CH = 128


def _sig(x):
    return 0.5 * jnp.tanh(0.5 * x) + 0.5


def _prep_body(kd_ref, ekp_ref, disc_ref, semb_ref, tbl_ref, ssemb_ref):
    tbl_ref[:, 0:_K] = _sig(kd_ref[...])
    tbl_ref[:, _EKP_COL:_EKP_COL + _K] = ekp_ref[...]
    n = kd_ref.shape[0]
    tbl_ref[:, _DISC_COL:_D] = jnp.zeros((n, _D - _DISC_COL), jnp.float32)
    tbl_ref[:, _DISC_COL:_DISC_COL + 1] = 10.0 * _sig(disc_ref[...])
    ssemb_ref[...] = _sig(semb_ref[...])


def _prep(k_difficulty, e_k_prob, e_discrimination, student_emb):
    exer_n, k = k_difficulty.shape
    stu_n = student_emb.shape[0]
    return pl.pallas_call(
        _prep_body,
        out_shape=[
            jax.ShapeDtypeStruct((exer_n, _D), jnp.float32),
            jax.ShapeDtypeStruct((stu_n, k), jnp.float32),
        ],
    )(k_difficulty, e_k_prob, e_discrimination, student_emb)


def _sc_gather(tbl, idx):
    B = idx.shape[0]
    D = tbl.shape[1]
    bpw = B // _NW
    mesh = plsc.VectorSubcoreMesh(core_axis_name="c", subcore_axis_name="s")

    @functools.partial(
        pl.kernel,
        mesh=mesh,
        out_type=jax.ShapeDtypeStruct((B, D), jnp.float32),
        scratch_types=[
            pltpu.VMEM((_CH,), jnp.int32),
            pltpu.VMEM((_CH, D), jnp.float32),
            pltpu.SemaphoreType.DMA,
        ],
    )
    def k(tbl_hbm, idx_hbm, out_hbm, idx_v, rows_v, sem):
        wid = lax.axis_index("s") * 2 + lax.axis_index("c")
        base = wid * bpw
        for ci in range(bpw // _CH):
            off = base + ci * _CH
            pltpu.sync_copy(idx_hbm.at[pl.ds(off, _CH)], idx_v)
            pltpu.async_copy(tbl_hbm.at[idx_v], rows_v, sem).wait()
            pltpu.sync_copy(rows_v, out_hbm.at[pl.ds(off, _CH)])

    return k(tbl, idx)


def _mlp_body(g_ref, m_ref, sid_ref, ssemb_ref,
              w1_ref, b1_ref, w2_ref, b2_ref, w3_ref, b3_ref,
              out_ref, ekp_ref):
    bb = m_ref.shape[0]
    stu_n = ssemb_ref.shape[0]
    ids = sid_ref[...]
    row = lax.broadcasted_iota(jnp.int32, (bb, stu_n), 1)
    oh = (ids == row).astype(jnp.float32)
    stat = jnp.dot(oh, ssemb_ref[...], preferred_element_type=jnp.float32)

    skd = g_ref[:, 0:_K]
    ekp = g_ref[:, _EKP_COL:_EKP_COL + _K]
    ekp_ref[...] = ekp
    disc = g_ref[:, _DISC_COL:_DISC_COL + 1]

    x = disc * (stat - skd) * (m_ref[...] * _sig(ekp))
    h1 = _sig(
        jnp.dot(x.astype(jnp.bfloat16), w1_ref[...].astype(jnp.bfloat16),
                preferred_element_type=jnp.float32)
        + b1_ref[...])
    h2 = _sig(
        jnp.dot(h1.astype(jnp.bfloat16), w2_ref[...].astype(jnp.bfloat16),
                preferred_element_type=jnp.float32)
        + b2_ref[...])
    p = _sig(
        jnp.dot(h2, w3_ref[...], preferred_element_type=jnp.float32)
        + b3_ref[...])
    out_ref[:, 0:1] = 1.0 - p
    out_ref[:, 1:2] = p


def _tc_mlp(gathered, masks, sid2, sig_semb, w1t, b1r, w2t, b2r, w3t, b3r):
    B = masks.shape[0]
    BB = 1024
    grid = (B // BB,)
    stu_n, k = sig_semb.shape
    l1 = w1t.shape[1]
    l2 = w2t.shape[1]
    full = lambda shp: pl.BlockSpec(shp, lambda i: (0, 0))
    return pl.pallas_call(
        _mlp_body,
        grid=grid,
        in_specs=[
            pl.BlockSpec((BB, _D), lambda i: (i, 0)),
            pl.BlockSpec((BB, k), lambda i: (i, 0)),
            pl.BlockSpec((BB, 1), lambda i: (i, 0)),
            full((stu_n, k)),
            full((k, l1)), full((1, l1)),
            full((l1, l2)), full((1, l2)),
            full((l2, 1)), full((1, 1)),
        ],
        out_specs=[
            pl.BlockSpec((BB, 2), lambda i: (i, 0)),
            pl.BlockSpec((BB, k), lambda i: (i, 0)),
        ],
        out_shape=[
            jax.ShapeDtypeStruct((B, 2), jnp.float32),
            jax.ShapeDtypeStruct((B, k), jnp.float32),
        ],
    )(gathered, masks, sid2, sig_semb,
      w1t, b1r, w2t, b2r, w3t, b3r)


def kernel(stu_id, input_exercise, knowledge_masks, student_emb, k_difficulty,
           e_discrimination, e_k_prob, W1, b1, W2, b2, W3, b3):
    tbl, sig_semb = _prep(k_difficulty, e_k_prob, e_discrimination,
                          student_emb)
    gathered = _sc_gather(tbl, input_exercise.astype(jnp.int32))
    out, ekp = _tc_mlp(
        gathered, knowledge_masks, stu_id.astype(jnp.int32).reshape(-1, 1),
        sig_semb, W1.T, b1.reshape(1, -1), W2.T, b2.reshape(1, -1),
        W3.T, b3.reshape(1, -1))
    return (out, ekp)


# speedup vs baseline: 1.6343x; 1.5279x over previous
import functools

import jax
import jax.numpy as jnp
from jax import lax
from jax.experimental import pallas as pl
from jax.experimental.pallas import tpu as pltpu
from jax.experimental.pallas import tpu_sc as plsc

_K = 197
_EKP_COL = 200
_DISC_COL = 400
_D = 512
_NW = 32
_CH = 128


def _sig(x):
    return 0.5 * jnp.tanh(0.5 * x) + 0.5


def _prep_body(kdT_ref, ekpT_ref, discT_ref, sembT_ref, tbl_ref, ssembT_ref):
    n = kdT_ref.shape[1]
    tbl_ref[:, 0:_K] = jnp.transpose(_sig(kdT_ref[...]))
    tbl_ref[:, _K:_EKP_COL] = jnp.zeros((n, _EKP_COL - _K), jnp.float32)
    tbl_ref[:, _EKP_COL:_EKP_COL + _K] = jnp.transpose(ekpT_ref[...])
    tbl_ref[:, _EKP_COL + _K:_D] = jnp.zeros((n, _D - _EKP_COL - _K),
                                             jnp.float32)
    tbl_ref[:, _DISC_COL:_DISC_COL + 1] = jnp.transpose(
        10.0 * _sig(discT_ref[...]))
    ssembT_ref[...] = _sig(sembT_ref[...])


def _prep(kdT, ekpT, discT, sembT):
    k, exer_n = kdT.shape
    stu_n = sembT.shape[1]
    return pl.pallas_call(
        _prep_body,
        out_shape=[
            jax.ShapeDtypeStruct((exer_n, _D), jnp.float32),
            jax.ShapeDtypeStruct((k, stu_n), jnp.float32),
        ],
    )(kdT, ekpT, discT, sembT)


def _sc_gather(tbl, idx):
    B = idx.shape[0]
    D = tbl.shape[1]
    bpw = B // _NW
    mesh = plsc.VectorSubcoreMesh(core_axis_name="c", subcore_axis_name="s")

    @functools.partial(
        pl.kernel,
        mesh=mesh,
        out_type=jax.ShapeDtypeStruct((B, D), jnp.float32),
        scratch_types=[
            pltpu.VMEM((_CH,), jnp.int32),
            pltpu.VMEM((_CH, D), jnp.float32),
            pltpu.SemaphoreType.DMA,
        ],
    )
    def k(tbl_hbm, idx_hbm, out_hbm, idx_v, rows_v, sem):
        wid = lax.axis_index("s") * 2 + lax.axis_index("c")
        base = wid * bpw
        for ci in range(bpw // _CH):
            off = base + ci * _CH
            pltpu.sync_copy(idx_hbm.at[pl.ds(off, _CH)], idx_v)
            pltpu.async_copy(tbl_hbm.at[idx_v], rows_v, sem).wait()
            pltpu.sync_copy(rows_v, out_hbm.at[pl.ds(off, _CH)])

    return k(tbl, idx)


def _mlp_body(g_ref, mT_ref, sid_ref, ssembT_ref,
              w1_ref, b1_ref, w2_ref, b2_ref, w3_ref, b3_ref,
              outT_ref, ekpT_ref):
    bb = g_ref.shape[0]
    stu_n = ssembT_ref.shape[1]
    ids = jnp.reshape(sid_ref[...], (1, bb))
    col = lax.broadcasted_iota(jnp.int32, (stu_n, bb), 0)
    ohT = (ids == col).astype(jnp.float32)
    statT = jnp.dot(ssembT_ref[...], ohT,
                    preferred_element_type=jnp.float32)

    gT = jnp.transpose(g_ref[...])
    skdT = gT[0:_K, :]
    ekpT = gT[_EKP_COL:_EKP_COL + _K, :]
    ekpT_ref[...] = ekpT
    discT = gT[_DISC_COL:_DISC_COL + 1, :]

    xT = discT * (statT - skdT) * (mT_ref[...] * _sig(ekpT))
    h1T = _sig(
        jnp.dot(w1_ref[...].astype(jnp.bfloat16), xT.astype(jnp.bfloat16),
                preferred_element_type=jnp.float32)
        + b1_ref[...])
    h2T = _sig(
        jnp.dot(w2_ref[...].astype(jnp.bfloat16), h1T.astype(jnp.bfloat16),
                preferred_element_type=jnp.float32)
        + b2_ref[...])
    pT = _sig(
        jnp.dot(w3_ref[...], h2T, preferred_element_type=jnp.float32)
        + b3_ref[...])
    outT_ref[0:1, :] = 1.0 - pT
    outT_ref[1:2, :] = pT


def _tc_mlp(gathered, masksT, sid, ssembT, w1, b1c, w2, b2c, w3, b3c):
    B = sid.shape[0]
    BB = 1024
    grid = (B // BB,)
    k, stu_n = ssembT.shape
    l1, l2 = w1.shape[0], w2.shape[0]
    full = lambda shp: pl.BlockSpec(shp, lambda i: (0, 0))
    return pl.pallas_call(
        _mlp_body,
        grid=grid,
        in_specs=[
            pl.BlockSpec((BB, _D), lambda i: (i, 0)),
            pl.BlockSpec((k, BB), lambda i: (0, i)),
            pl.BlockSpec((BB,), lambda i: (i,)),
            full((k, stu_n)),
            full((l1, k)), full((l1, 1)),
            full((l2, l1)), full((l2, 1)),
            full((1, l2)), full((1, 1)),
        ],
        out_specs=[
            pl.BlockSpec((2, BB), lambda i: (0, i)),
            pl.BlockSpec((k, BB), lambda i: (0, i)),
        ],
        out_shape=[
            jax.ShapeDtypeStruct((2, B), jnp.float32),
            jax.ShapeDtypeStruct((k, B), jnp.float32),
        ],
    )(gathered, masksT, sid, ssembT, w1, b1c, w2, b2c, w3, b3c)


def kernel(stu_id, input_exercise, knowledge_masks, student_emb, k_difficulty,
           e_discrimination, e_k_prob, W1, b1, W2, b2, W3, b3):
    tbl, ssembT = _prep(k_difficulty.T, e_k_prob.T, e_discrimination.T,
                        student_emb.T)
    gathered = _sc_gather(tbl, input_exercise.astype(jnp.int32))
    outT, ekpT = _tc_mlp(
        gathered, knowledge_masks.T, stu_id.astype(jnp.int32),
        ssembT, W1, b1.reshape(-1, 1), W2, b2.reshape(-1, 1),
        W3, b3.reshape(-1, 1))
    return (outT.T, ekpT.T)


# speedup vs baseline: 1.6572x; 1.0140x over previous
import functools

import jax
import jax.numpy as jnp
from jax import lax
from jax.experimental import pallas as pl
from jax.experimental.pallas import tpu as pltpu
from jax.experimental.pallas import tpu_sc as plsc

_K = 197
_EKP_COL = 200
_DISC_COL = 400
_D = 512
_NW = 32
_CH = 64


def _sig(x):
    return 0.5 * jnp.tanh(0.5 * x) + 0.5


def _prep_body(kdT_ref, ekpT_ref, discT_ref, sembT_ref, tbl_ref, ssembT_ref):
    n = kdT_ref.shape[1]
    tbl_ref[:, 0:_K] = jnp.transpose(_sig(kdT_ref[...]))
    tbl_ref[:, _K:_EKP_COL] = jnp.zeros((n, _EKP_COL - _K), jnp.float32)
    tbl_ref[:, _EKP_COL:_EKP_COL + _K] = jnp.transpose(ekpT_ref[...])
    tbl_ref[:, _EKP_COL + _K:_D] = jnp.zeros((n, _D - _EKP_COL - _K),
                                             jnp.float32)
    tbl_ref[:, _DISC_COL:_DISC_COL + 1] = jnp.transpose(
        10.0 * _sig(discT_ref[...]))
    ssembT_ref[...] = _sig(sembT_ref[...])


def _prep(kdT, ekpT, discT, sembT):
    k, exer_n = kdT.shape
    stu_n = sembT.shape[1]
    return pl.pallas_call(
        _prep_body,
        out_shape=[
            jax.ShapeDtypeStruct((exer_n, _D), jnp.float32),
            jax.ShapeDtypeStruct((k, stu_n), jnp.float32),
        ],
    )(kdT, ekpT, discT, sembT)


def _sc_gather(tbl, idx):
    B = idx.shape[0]
    D = tbl.shape[1]
    bpw = B // _NW
    mesh = plsc.VectorSubcoreMesh(core_axis_name="c", subcore_axis_name="s")

    nch = bpw // _CH

    @functools.partial(
        pl.kernel,
        mesh=mesh,
        out_type=jax.ShapeDtypeStruct((B, D), jnp.float32),
        scratch_types=[
            pltpu.VMEM((_CH,), jnp.int32),
            pltpu.VMEM((_CH,), jnp.int32),
            pltpu.VMEM((_CH, D), jnp.float32),
            pltpu.VMEM((_CH, D), jnp.float32),
            pltpu.SemaphoreType.DMA,
            pltpu.SemaphoreType.DMA,
            pltpu.SemaphoreType.DMA,
            pltpu.SemaphoreType.DMA,
        ],
    )
    def k(tbl_hbm, idx_hbm, out_hbm, idx_v0, idx_v1, r0, r1,
          g0, g1, s0, s1):
        wid = lax.axis_index("s") * 2 + lax.axis_index("c")
        base = wid * bpw
        idx_v = [idx_v0, idx_v1]
        rows = [r0, r1]
        gsem = [g0, g1]
        ssem = [s0, s1]
        gh = [None, None]
        sh = [None, None]
        for b in range(min(2, nch)):
            pltpu.sync_copy(idx_hbm.at[pl.ds(base + b * _CH, _CH)], idx_v[b])
            gh[b] = pltpu.async_copy(tbl_hbm.at[idx_v[b]], rows[b], gsem[b])
        for ci in range(nch):
            b = ci % 2
            gh[b].wait()
            sh[b] = pltpu.async_copy(
                rows[b], out_hbm.at[pl.ds(base + ci * _CH, _CH)], ssem[b])
            if ci + 2 < nch:
                sh[b].wait()
                pltpu.sync_copy(
                    idx_hbm.at[pl.ds(base + (ci + 2) * _CH, _CH)], idx_v[b])
                gh[b] = pltpu.async_copy(tbl_hbm.at[idx_v[b]], rows[b],
                                         gsem[b])
        for b in range(min(2, nch)):
            if sh[b] is not None:
                sh[b].wait()

    return k(tbl, idx)


def _mlp_body(g_ref, mT_ref, sid_ref, ssembT_ref,
              w1_ref, b1_ref, w2_ref, b2_ref, w3_ref, b3_ref,
              outT_ref, ekpT_ref):
    bb = g_ref.shape[0]
    stu_n = ssembT_ref.shape[1]
    ids = jnp.reshape(sid_ref[...], (1, bb))
    col = lax.broadcasted_iota(jnp.int32, (stu_n, bb), 0)
    ohT = (ids == col).astype(jnp.float32)
    statT = jnp.dot(ssembT_ref[...], ohT,
                    preferred_element_type=jnp.float32)

    gT = jnp.transpose(g_ref[...])
    skdT = gT[0:_K, :]
    ekpT = gT[_EKP_COL:_EKP_COL + _K, :]
    ekpT_ref[...] = ekpT
    discT = gT[_DISC_COL:_DISC_COL + 1, :]

    xT = discT * (statT - skdT) * (mT_ref[...] * _sig(ekpT))
    h1T = _sig(
        jnp.dot(w1_ref[...].astype(jnp.bfloat16), xT.astype(jnp.bfloat16),
                preferred_element_type=jnp.float32)
        + b1_ref[...])
    h2T = _sig(
        jnp.dot(w2_ref[...].astype(jnp.bfloat16), h1T.astype(jnp.bfloat16),
                preferred_element_type=jnp.float32)
        + b2_ref[...])
    pT = _sig(
        jnp.dot(w3_ref[...], h2T, preferred_element_type=jnp.float32)
        + b3_ref[...])
    outT_ref[0:1, :] = 1.0 - pT
    outT_ref[1:2, :] = pT


def _tc_mlp(gathered, masksT, sid, ssembT, w1, b1c, w2, b2c, w3, b3c):
    B = sid.shape[0]
    BB = 1024
    grid = (B // BB,)
    k, stu_n = ssembT.shape
    l1, l2 = w1.shape[0], w2.shape[0]
    full = lambda shp: pl.BlockSpec(shp, lambda i: (0, 0))
    return pl.pallas_call(
        _mlp_body,
        grid=grid,
        in_specs=[
            pl.BlockSpec((BB, _D), lambda i: (i, 0)),
            pl.BlockSpec((k, BB), lambda i: (0, i)),
            pl.BlockSpec((BB,), lambda i: (i,)),
            full((k, stu_n)),
            full((l1, k)), full((l1, 1)),
            full((l2, l1)), full((l2, 1)),
            full((1, l2)), full((1, 1)),
        ],
        out_specs=[
            pl.BlockSpec((2, BB), lambda i: (0, i)),
            pl.BlockSpec((k, BB), lambda i: (0, i)),
        ],
        out_shape=[
            jax.ShapeDtypeStruct((2, B), jnp.float32),
            jax.ShapeDtypeStruct((k, B), jnp.float32),
        ],
    )(gathered, masksT, sid, ssembT, w1, b1c, w2, b2c, w3, b3c)


def kernel(stu_id, input_exercise, knowledge_masks, student_emb, k_difficulty,
           e_discrimination, e_k_prob, W1, b1, W2, b2, W3, b3):
    tbl, ssembT = _prep(k_difficulty.T, e_k_prob.T, e_discrimination.T,
                        student_emb.T)
    gathered = _sc_gather(tbl, input_exercise.astype(jnp.int32))
    outT, ekpT = _tc_mlp(
        gathered, knowledge_masks.T, stu_id.astype(jnp.int32),
        ssembT, W1, b1.reshape(-1, 1), W2, b2.reshape(-1, 1),
        W3, b3.reshape(-1, 1))
    return (outT.T, ekpT.T)
